# Initial kernel scaffold; baseline (speedup 1.0000x reference)
import jax
import jax.numpy as jnp
from jax.experimental import pallas as pl


def kernel(init_embed, init_rel, W, W_loop, W_rel, b, edge_index, edge_type, sub, rel):
    raise NotImplementedError("write your pallas kernel here")


# speedup vs baseline: 7.0025x; 7.0025x over previous
import functools

import jax
import jax.numpy as jnp
from jax import lax
from jax.experimental import pallas as pl
from jax.experimental.pallas import tpu as pltpu
from jax.experimental.pallas import tpu_sc as plsc

N_ENT = 10000
D = 128
EXT = 144
E = 320000
B = 4096
R2 = 200

_info = plsc.get_sparse_core_info()
NC = _info.num_cores
NS = _info.num_subcores
NW = NC * NS
EPW = E // NW
CHUNK = 100
NJ = EPW // CHUNK
JBLK = 10
NBLK = NJ // JBLK
N_PAD = 10240
RPT = N_PAD // NS

_mesh = plsc.VectorSubcoreMesh(core_axis_name="c", subcore_axis_name="s")


@functools.partial(
    pl.kernel,
    mesh=_mesh,
    compiler_params=pltpu.CompilerParams(use_tc_tiling_on_sc=False),
    out_type=jax.ShapeDtypeStruct((NC, N_PAD, EXT), jnp.float32),
    scratch_types=[
        pltpu.VMEM((JBLK, CHUNK), jnp.int32),
        pltpu.VMEM((JBLK, CHUNK), jnp.int32),
        pltpu.VMEM((JBLK, CHUNK), jnp.int32),
        pltpu.VMEM((CHUNK, EXT), jnp.float32),
        pltpu.VMEM((CHUNK, EXT), jnp.float32),
        pltpu.VMEM_SHARED((N_PAD, EXT), jnp.float32),
        pltpu.SemaphoreType.DMA,
        pltpu.SemaphoreType.DMA,
    ],
)
def _sc_aggregate(embed_ext_hbm, negrel_ext_hbm, src_hbm, dst_hbm, typ_hbm,
                  zeros_hbm, out_hbm,
                  src_v, dst_v, typ_v, buf_a, buf_b, acc, sem_a, sem_b):
    cid = lax.axis_index("c")
    sid = lax.axis_index("s")
    wid = cid * NS + sid
    pltpu.sync_copy(zeros_hbm, acc.at[pl.ds(sid * RPT, RPT)])
    plsc.subcore_barrier()

    def block(t, carry):
        pltpu.sync_copy(src_hbm.at[wid, pl.ds(t * JBLK, JBLK)], src_v)
        pltpu.sync_copy(dst_hbm.at[wid, pl.ds(t * JBLK, JBLK)], dst_v)
        pltpu.sync_copy(typ_hbm.at[wid, pl.ds(t * JBLK, JBLK)], typ_v)

        def body(j, c2):
            cp_a = pltpu.async_copy(embed_ext_hbm.at[src_v.at[j]], buf_a, sem_a)
            cp_b = pltpu.async_copy(negrel_ext_hbm.at[typ_v.at[j]], buf_b, sem_b)
            cp_a.wait()
            pltpu.sync_copy(buf_a, acc.at[dst_v.at[j]], add=True)
            cp_b.wait()
            pltpu.sync_copy(buf_b, acc.at[dst_v.at[j]], add=True)
            return c2

        lax.fori_loop(0, JBLK, body, 0)
        return carry

    lax.fori_loop(0, NBLK, block, 0)
    plsc.subcore_barrier()
    pltpu.sync_copy(acc.at[pl.ds(sid * RPT, RPT)],
                    out_hbm.at[cid, pl.ds(sid * RPT, RPT)])


BPW = B // NW


@functools.partial(
    pl.kernel,
    mesh=_mesh,
    out_type=(jax.ShapeDtypeStruct((B, D), jnp.float32),
              jax.ShapeDtypeStruct((B, D), jnp.float32)),
    scratch_types=[
        pltpu.VMEM((BPW,), jnp.int32),
        pltpu.VMEM((BPW,), jnp.int32),
        pltpu.VMEM((BPW, D), jnp.float32),
        pltpu.VMEM((BPW, D), jnp.float32),
        pltpu.SemaphoreType.DMA,
        pltpu.SemaphoreType.DMA,
    ],
)
def _sc_lookup(x_hbm, r_hbm, sub_hbm, rel_hbm, sub_out, rel_out,
               sub_v, rel_v, buf_x, buf_r, sem_x, sem_r):
    wid = lax.axis_index("c") * NS + lax.axis_index("s")
    base = wid * BPW
    pltpu.sync_copy(sub_hbm.at[pl.ds(base, BPW)], sub_v)
    pltpu.sync_copy(rel_hbm.at[pl.ds(base, BPW)], rel_v)
    cp_x = pltpu.async_copy(x_hbm.at[sub_v], buf_x, sem_x)
    cp_r = pltpu.async_copy(r_hbm.at[rel_v], buf_r, sem_r)
    cp_x.wait()
    cp_r.wait()
    pltpu.sync_copy(buf_x, sub_out.at[pl.ds(base, BPW)])
    pltpu.sync_copy(buf_r, rel_out.at[pl.ds(base, BPW)])


def _tc_dense(acc_ref, embed_ref, w_ref, wl_ref, b_ref, rel_ref, wr_ref,
              x_ref, r_ref):
    pre = acc_ref[0, :N_ENT, :D] + acc_ref[1, :N_ENT, :D]
    deg = acc_ref[0, :N_ENT, D:D + 1] + acc_ref[1, :N_ENT, D:D + 1]
    norm = 1.0 / jnp.maximum(deg, 1.0)
    agg = jnp.dot(pre, w_ref[...], preferred_element_type=jnp.float32) * norm
    loop = jnp.dot(embed_ref[...], wl_ref[...], preferred_element_type=jnp.float32)
    x_ref[...] = jnp.tanh(agg + loop + b_ref[...])
    r_ref[...] = jnp.dot(rel_ref[...], wr_ref[...], preferred_element_type=jnp.float32)


_tc_dense_call = pl.pallas_call(
    _tc_dense,
    out_shape=(jax.ShapeDtypeStruct((N_ENT, D), jnp.float32),
               jax.ShapeDtypeStruct((R2, D), jnp.float32)),
)


def kernel(init_embed, init_rel, W, W_loop, W_rel, b, edge_index, edge_type,
           sub, rel):
    f32 = jnp.float32
    src = edge_index[0].astype(jnp.int32).reshape(NW, NJ, CHUNK)
    dst = edge_index[1].astype(jnp.int32).reshape(NW, NJ, CHUNK)
    typ = edge_type.astype(jnp.int32).reshape(NW, NJ, CHUNK)
    embed_ext = jnp.concatenate(
        [init_embed.astype(f32),
         jnp.ones((N_ENT, 1), f32),
         jnp.zeros((N_ENT, EXT - D - 1), f32)], axis=1)
    negrel_ext = jnp.concatenate(
        [-init_rel.astype(f32), jnp.zeros((R2, EXT - D), f32)], axis=1)
    zeros_blk = jnp.zeros((RPT, EXT), f32)

    acc2 = _sc_aggregate(embed_ext, negrel_ext, src, dst, typ, zeros_blk)
    x, r = _tc_dense_call(acc2, init_embed, W, W_loop,
                          b.reshape(1, D), init_rel, W_rel)
    sub_emb, rel_emb = _sc_lookup(x, r, sub.astype(jnp.int32),
                                  rel.astype(jnp.int32))
    return (sub_emb, rel_emb, x)


# speedup vs baseline: 7.0796x; 1.0110x over previous
import functools

import jax
import jax.numpy as jnp
from jax import lax
from jax.experimental import pallas as pl
from jax.experimental.pallas import tpu as pltpu
from jax.experimental.pallas import tpu_sc as plsc

N_ENT = 10000
D = 128
EXT = 144
E = 320000
B = 4096
R2 = 200

_info = plsc.get_sparse_core_info()
NC = _info.num_cores
NS = _info.num_subcores
NW = NC * NS
EPW = E // NW
CHUNK = 50
NJ = EPW // CHUNK
JBLK = 20
NBLK = NJ // JBLK
N_PAD = 10240
RPT = N_PAD // NS

_mesh = plsc.VectorSubcoreMesh(core_axis_name="c", subcore_axis_name="s")


@functools.partial(
    pl.kernel,
    mesh=_mesh,
    compiler_params=pltpu.CompilerParams(use_tc_tiling_on_sc=False),
    out_type=jax.ShapeDtypeStruct((NC, N_PAD, EXT), jnp.float32),
    scratch_types=[
        pltpu.VMEM((JBLK, CHUNK), jnp.int32),
        pltpu.VMEM((JBLK, CHUNK), jnp.int32),
        pltpu.VMEM((JBLK, CHUNK), jnp.int32),
        pltpu.VMEM((CHUNK, EXT), jnp.float32),
        pltpu.VMEM((CHUNK, EXT), jnp.float32),
        pltpu.VMEM((CHUNK, EXT), jnp.float32),
        pltpu.VMEM((CHUNK, EXT), jnp.float32),
        pltpu.VMEM_SHARED((N_PAD, EXT), jnp.float32),
        pltpu.SemaphoreType.DMA,
        pltpu.SemaphoreType.DMA,
        pltpu.SemaphoreType.DMA,
        pltpu.SemaphoreType.DMA,
        pltpu.SemaphoreType.DMA,
        pltpu.SemaphoreType.DMA,
        pltpu.SemaphoreType.DMA,
        pltpu.SemaphoreType.DMA,
    ],
)
def _sc_aggregate(embed_ext_hbm, negrel_ext_hbm, eidx_hbm, typ_hbm,
                  zeros_hbm, out_hbm,
                  src_v, dst_v, typ_v, buf_a0, buf_a1, buf_b0, buf_b1, acc,
                  sga0, sga1, sgb0, sgb1, ssc0, ssc1, ssc2, ssc3):
    cid = lax.axis_index("c")
    sid = lax.axis_index("s")
    wid = cid * NS + sid
    pltpu.sync_copy(zeros_hbm, acc.at[pl.ds(sid * RPT, RPT)])
    plsc.subcore_barrier()

    def block(t, carry):
        pltpu.sync_copy(eidx_hbm.at[0, wid, pl.ds(t * JBLK, JBLK)], src_v)
        pltpu.sync_copy(eidx_hbm.at[1, wid, pl.ds(t * JBLK, JBLK)], dst_v)
        pltpu.sync_copy(typ_hbm.at[wid, pl.ds(t * JBLK, JBLK)], typ_v)

        def pair(p, c2):
            j0 = 2 * p
            j1 = 2 * p + 1
            ga0 = pltpu.async_copy(embed_ext_hbm.at[src_v.at[j0]], buf_a0, sga0)
            gb0 = pltpu.async_copy(negrel_ext_hbm.at[typ_v.at[j0]], buf_b0, sgb0)
            ga1 = pltpu.async_copy(embed_ext_hbm.at[src_v.at[j1]], buf_a1, sga1)
            gb1 = pltpu.async_copy(negrel_ext_hbm.at[typ_v.at[j1]], buf_b1, sgb1)
            ga0.wait()
            sc0 = pltpu.async_copy(buf_a0, acc.at[dst_v.at[j0]], ssc0, add=True)
            gb0.wait()
            sc1 = pltpu.async_copy(buf_b0, acc.at[dst_v.at[j0]], ssc1, add=True)
            ga1.wait()
            sc2 = pltpu.async_copy(buf_a1, acc.at[dst_v.at[j1]], ssc2, add=True)
            gb1.wait()
            sc3 = pltpu.async_copy(buf_b1, acc.at[dst_v.at[j1]], ssc3, add=True)
            sc0.wait()
            sc1.wait()
            sc2.wait()
            sc3.wait()
            return c2

        lax.fori_loop(0, JBLK // 2, pair, 0)
        return carry

    lax.fori_loop(0, NBLK, block, 0)
    plsc.subcore_barrier()
    pltpu.sync_copy(acc.at[pl.ds(sid * RPT, RPT)],
                    out_hbm.at[cid, pl.ds(sid * RPT, RPT)])


BPW = B // NW


@functools.partial(
    pl.kernel,
    mesh=_mesh,
    compiler_params=pltpu.CompilerParams(use_tc_tiling_on_sc=False),
    out_type=(jax.ShapeDtypeStruct((B, D), jnp.float32),
              jax.ShapeDtypeStruct((B, D), jnp.float32)),
    scratch_types=[
        pltpu.VMEM((BPW,), jnp.int32),
        pltpu.VMEM((BPW,), jnp.int32),
        pltpu.VMEM((BPW, D), jnp.float32),
        pltpu.VMEM((BPW, D), jnp.float32),
        pltpu.SemaphoreType.DMA,
        pltpu.SemaphoreType.DMA,
    ],
)
def _sc_lookup(x_hbm, r_hbm, sub_hbm, rel_hbm, sub_out, rel_out,
               sub_v, rel_v, buf_x, buf_r, sem_x, sem_r):
    wid = lax.axis_index("c") * NS + lax.axis_index("s")
    base = wid * BPW
    pltpu.sync_copy(sub_hbm.at[pl.ds(base, BPW)], sub_v)
    pltpu.sync_copy(rel_hbm.at[pl.ds(base, BPW)], rel_v)
    cp_x = pltpu.async_copy(x_hbm.at[sub_v], buf_x, sem_x)
    cp_r = pltpu.async_copy(r_hbm.at[rel_v], buf_r, sem_r)
    cp_x.wait()
    cp_r.wait()
    pltpu.sync_copy(buf_x, sub_out.at[pl.ds(base, BPW)])
    pltpu.sync_copy(buf_r, rel_out.at[pl.ds(base, BPW)])


def _tc_dense(acc_ref, embed_ref, w_ref, wl_ref, b_ref, rel_ref, wr_ref,
              x_ref, r_ref):
    pre = acc_ref[0, :N_ENT, :D] + acc_ref[1, :N_ENT, :D]
    deg = acc_ref[0, :N_ENT, D:D + 1] + acc_ref[1, :N_ENT, D:D + 1]
    norm = 1.0 / jnp.maximum(deg, 1.0)
    agg = jnp.dot(pre, w_ref[...], preferred_element_type=jnp.float32) * norm
    loop = jnp.dot(embed_ref[...], wl_ref[...], preferred_element_type=jnp.float32)
    x_ref[...] = jnp.tanh(agg + loop + b_ref[...])
    r_ref[...] = jnp.dot(rel_ref[...], wr_ref[...], preferred_element_type=jnp.float32)


_tc_dense_call = pl.pallas_call(
    _tc_dense,
    out_shape=(jax.ShapeDtypeStruct((N_ENT, D), jnp.float32),
               jax.ShapeDtypeStruct((R2, D), jnp.float32)),
)


def kernel(init_embed, init_rel, W, W_loop, W_rel, b, edge_index, edge_type,
           sub, rel):
    f32 = jnp.float32
    eidx = edge_index.astype(jnp.int32).reshape(2, NW, NJ, CHUNK)
    typ = edge_type.astype(jnp.int32).reshape(NW, NJ, CHUNK)
    embed_ext = jnp.concatenate(
        [init_embed.astype(f32),
         jnp.ones((N_ENT, 1), f32),
         jnp.zeros((N_ENT, EXT - D - 1), f32)], axis=1)
    negrel_ext = jnp.concatenate(
        [-init_rel.astype(f32), jnp.zeros((R2, EXT - D), f32)], axis=1)
    zeros_blk = jnp.zeros((RPT, EXT), f32)

    acc2 = _sc_aggregate(embed_ext, negrel_ext, eidx, typ, zeros_blk)
    x, r = _tc_dense_call(acc2, init_embed, W, W_loop,
                          b.reshape(1, D), init_rel, W_rel)
    sub_emb, rel_emb = _sc_lookup(x, r, sub.astype(jnp.int32),
                                  rel.astype(jnp.int32))
    return (sub_emb, rel_emb, x)


# speedup vs baseline: 7.1688x; 1.0126x over previous
import functools

import jax
import jax.numpy as jnp
from jax import lax
from jax.experimental import pallas as pl
from jax.experimental.pallas import tpu as pltpu
from jax.experimental.pallas import tpu_sc as plsc

N_ENT = 10000
D = 128
EXT = 144
E = 320000
B = 4096
R2 = 200

_info = plsc.get_sparse_core_info()
NC = _info.num_cores
NS = _info.num_subcores
NW = NC * NS
EPW = E // NW
CHUNK = 50
NJ = EPW // CHUNK
JBLK = 20
NBLK = NJ // JBLK
N_PAD = 10240
RPT = N_PAD // NS

_mesh = plsc.VectorSubcoreMesh(core_axis_name="c", subcore_axis_name="s")


@functools.partial(
    pl.kernel,
    mesh=_mesh,
    compiler_params=pltpu.CompilerParams(use_tc_tiling_on_sc=False),
    out_type=jax.ShapeDtypeStruct((NC, N_PAD, EXT), jnp.float32),
    scratch_types=[
        pltpu.VMEM((JBLK, CHUNK), jnp.int32),
        pltpu.VMEM((JBLK, CHUNK), jnp.int32),
        pltpu.VMEM((JBLK, CHUNK), jnp.int32),
        pltpu.VMEM((CHUNK, EXT), jnp.float32),
        pltpu.VMEM((CHUNK, EXT), jnp.float32),
        pltpu.VMEM((CHUNK, EXT), jnp.float32),
        pltpu.VMEM((CHUNK, EXT), jnp.float32),
        pltpu.VMEM_SHARED((N_PAD, EXT), jnp.float32),
        pltpu.SemaphoreType.DMA,
        pltpu.SemaphoreType.DMA,
        pltpu.SemaphoreType.DMA,
        pltpu.SemaphoreType.DMA,
        pltpu.SemaphoreType.DMA,
        pltpu.SemaphoreType.DMA,
        pltpu.SemaphoreType.DMA,
        pltpu.SemaphoreType.DMA,
    ],
)
def _sc_aggregate(embed_ext_hbm, negrel_ext_hbm, eidx_hbm, typ_hbm,
                  zeros_hbm, out_hbm,
                  src_v, dst_v, typ_v, buf_a0, buf_a1, buf_b0, buf_b1, acc,
                  sga0, sga1, sgb0, sgb1, ssc0, ssc1, ssc2, ssc3):
    cid = lax.axis_index("c")
    sid = lax.axis_index("s")
    wid = cid * NS + sid
    pltpu.sync_copy(zeros_hbm, acc.at[pl.ds(sid * RPT, RPT)])
    plsc.subcore_barrier()

    def block(t, carry):
        pltpu.sync_copy(eidx_hbm.at[0, wid, pl.ds(t * JBLK, JBLK)], src_v)
        pltpu.sync_copy(eidx_hbm.at[1, wid, pl.ds(t * JBLK, JBLK)], dst_v)
        pltpu.sync_copy(typ_hbm.at[wid, pl.ds(t * JBLK, JBLK)], typ_v)

        def pair(p, c2):
            j0 = 2 * p
            j1 = 2 * p + 1
            ga0 = pltpu.async_copy(embed_ext_hbm.at[src_v.at[j0]], buf_a0, sga0)
            ga1 = pltpu.async_copy(embed_ext_hbm.at[src_v.at[j1]], buf_a1, sga1)
            ga0.wait()
            gb0 = pltpu.async_copy(negrel_ext_hbm.at[typ_v.at[j0]], buf_a0,
                                   sgb0, add=True)
            ga1.wait()
            gb1 = pltpu.async_copy(negrel_ext_hbm.at[typ_v.at[j1]], buf_a1,
                                   sgb1, add=True)
            gb0.wait()
            sc0 = pltpu.async_copy(buf_a0, acc.at[dst_v.at[j0]], ssc0, add=True)
            gb1.wait()
            sc1 = pltpu.async_copy(buf_a1, acc.at[dst_v.at[j1]], ssc1, add=True)
            sc0.wait()
            sc1.wait()
            return c2

        lax.fori_loop(0, JBLK // 2, pair, 0)
        return carry

    lax.fori_loop(0, NBLK, block, 0)
    plsc.subcore_barrier()
    pltpu.sync_copy(acc.at[pl.ds(sid * RPT, RPT)],
                    out_hbm.at[cid, pl.ds(sid * RPT, RPT)])


BPW = B // NW


@functools.partial(
    pl.kernel,
    mesh=_mesh,
    compiler_params=pltpu.CompilerParams(use_tc_tiling_on_sc=False),
    out_type=(jax.ShapeDtypeStruct((B, D), jnp.float32),
              jax.ShapeDtypeStruct((B, D), jnp.float32)),
    scratch_types=[
        pltpu.VMEM((BPW,), jnp.int32),
        pltpu.VMEM((BPW,), jnp.int32),
        pltpu.VMEM((BPW, D), jnp.float32),
        pltpu.VMEM((BPW, D), jnp.float32),
        pltpu.SemaphoreType.DMA,
        pltpu.SemaphoreType.DMA,
    ],
)
def _sc_lookup(x_hbm, r_hbm, sub_hbm, rel_hbm, sub_out, rel_out,
               sub_v, rel_v, buf_x, buf_r, sem_x, sem_r):
    wid = lax.axis_index("c") * NS + lax.axis_index("s")
    base = wid * BPW
    pltpu.sync_copy(sub_hbm.at[pl.ds(base, BPW)], sub_v)
    pltpu.sync_copy(rel_hbm.at[pl.ds(base, BPW)], rel_v)
    cp_x = pltpu.async_copy(x_hbm.at[sub_v], buf_x, sem_x)
    cp_r = pltpu.async_copy(r_hbm.at[rel_v], buf_r, sem_r)
    cp_x.wait()
    cp_r.wait()
    pltpu.sync_copy(buf_x, sub_out.at[pl.ds(base, BPW)])
    pltpu.sync_copy(buf_r, rel_out.at[pl.ds(base, BPW)])


def _tc_dense(acc_ref, embed_ref, w_ref, wl_ref, b_ref, rel_ref, wr_ref,
              x_ref, r_ref):
    pre = acc_ref[0, :N_ENT, :D] + acc_ref[1, :N_ENT, :D]
    deg = acc_ref[0, :N_ENT, D:D + 1] + acc_ref[1, :N_ENT, D:D + 1]
    norm = 1.0 / jnp.maximum(deg, 1.0)
    agg = jnp.dot(pre, w_ref[...], preferred_element_type=jnp.float32) * norm
    loop = jnp.dot(embed_ref[...], wl_ref[...], preferred_element_type=jnp.float32)
    x_ref[...] = jnp.tanh(agg + loop + b_ref[...])
    r_ref[...] = jnp.dot(rel_ref[...], wr_ref[...], preferred_element_type=jnp.float32)


_tc_dense_call = pl.pallas_call(
    _tc_dense,
    out_shape=(jax.ShapeDtypeStruct((N_ENT, D), jnp.float32),
               jax.ShapeDtypeStruct((R2, D), jnp.float32)),
)


def kernel(init_embed, init_rel, W, W_loop, W_rel, b, edge_index, edge_type,
           sub, rel):
    f32 = jnp.float32
    eidx = edge_index.astype(jnp.int32).reshape(2, NW, NJ, CHUNK)
    typ = edge_type.astype(jnp.int32).reshape(NW, NJ, CHUNK)
    embed_ext = jnp.concatenate(
        [init_embed.astype(f32),
         jnp.ones((N_ENT, 1), f32),
         jnp.zeros((N_ENT, EXT - D - 1), f32)], axis=1)
    negrel_ext = jnp.concatenate(
        [-init_rel.astype(f32), jnp.zeros((R2, EXT - D), f32)], axis=1)
    zeros_blk = jnp.zeros((RPT, EXT), f32)

    acc2 = _sc_aggregate(embed_ext, negrel_ext, eidx, typ, zeros_blk)
    x, r = _tc_dense_call(acc2, init_embed, W, W_loop,
                          b.reshape(1, D), init_rel, W_rel)
    sub_emb, rel_emb = _sc_lookup(x, r, sub.astype(jnp.int32),
                                  rel.astype(jnp.int32))
    return (sub_emb, rel_emb, x)


# speedup vs baseline: 7.7520x; 1.0814x over previous
import functools

import jax
import jax.numpy as jnp
from jax import lax
from jax.experimental import pallas as pl
from jax.experimental.pallas import tpu as pltpu
from jax.experimental.pallas import tpu_sc as plsc

N_ENT = 10000
D = 128
EXT = 144
E = 320000
B = 4096
R2 = 200

_info = plsc.get_sparse_core_info()
NC = _info.num_cores
NS = _info.num_subcores
NW = NC * NS
EPW = E // NW
CHUNK = 100
NJ = EPW // CHUNK
JBLK = 10
NBLK = NJ // JBLK
N_PAD = 10240
RPT = N_PAD // NS

_mesh = plsc.VectorSubcoreMesh(core_axis_name="c", subcore_axis_name="s")


@functools.partial(
    pl.kernel,
    mesh=_mesh,
    compiler_params=pltpu.CompilerParams(use_tc_tiling_on_sc=False),
    out_type=jax.ShapeDtypeStruct((NC, N_PAD, EXT), jnp.float32),
    scratch_types=[
        pltpu.VMEM((2, JBLK, 3, CHUNK), jnp.int32),
        pltpu.VMEM((CHUNK, EXT), jnp.float32),
        pltpu.VMEM((CHUNK, EXT), jnp.float32),
        pltpu.VMEM_SHARED((N_PAD, EXT), jnp.float32),
        pltpu.SemaphoreType.DMA,
        pltpu.SemaphoreType.DMA,
        pltpu.SemaphoreType.DMA,
        pltpu.SemaphoreType.DMA,
        pltpu.SemaphoreType.DMA,
        pltpu.SemaphoreType.DMA,
    ],
)
def _sc_aggregate(embed_ext_hbm, negrel_ext_hbm, idx_hbm, zeros_hbm, out_hbm,
                  idx_v, buf0, buf1, acc,
                  sga0, sga1, sgb0, sgb1, ssc0, ssc1):
    cid = lax.axis_index("c")
    sid = lax.axis_index("s")
    wid = cid * NS + sid
    bufs = (buf0, buf1)
    sgas = (sga0, sga1)
    sgbs = (sgb0, sgb1)
    sscs = (ssc0, ssc1)
    pltpu.sync_copy(zeros_hbm, acc.at[pl.ds(sid * RPT, RPT)])
    plsc.subcore_barrier()

    def block(t, carry):
        tb = lax.rem(t, 2)
        pltpu.sync_copy(idx_hbm.at[wid, pl.ds(t * JBLK, JBLK)], idx_v.at[tb])

        def pair(p, c2):
            nonfirst = jnp.logical_or(t > 0, p > 0)
            for s in range(2):
                j = 2 * p + s

                @pl.when(nonfirst)
                def _drain(s=s, j=j):
                    pltpu.make_async_copy(
                        bufs[s], acc.at[idx_v.at[tb, j, 1]], sscs[s]).wait()

                pltpu.async_copy(embed_ext_hbm.at[idx_v.at[tb, j, 0]],
                                 bufs[s], sgas[s])
            for s in range(2):
                j = 2 * p + s
                pltpu.make_async_copy(embed_ext_hbm.at[idx_v.at[tb, j, 0]],
                                      bufs[s], sgas[s]).wait()
                pltpu.async_copy(negrel_ext_hbm.at[idx_v.at[tb, j, 2]],
                                 bufs[s], sgbs[s], add=True)
            for s in range(2):
                j = 2 * p + s
                pltpu.make_async_copy(negrel_ext_hbm.at[idx_v.at[tb, j, 2]],
                                      bufs[s], sgbs[s]).wait()
                pltpu.async_copy(bufs[s], acc.at[idx_v.at[tb, j, 1]],
                                 sscs[s], add=True)
            return c2

        lax.fori_loop(0, JBLK // 2, pair, 0)
        return carry

    lax.fori_loop(0, NBLK, block, 0)
    for s in range(2):
        pltpu.make_async_copy(bufs[s], acc.at[idx_v.at[0, 0, 1]],
                              sscs[s]).wait()
    plsc.subcore_barrier()
    pltpu.sync_copy(acc.at[pl.ds(sid * RPT, RPT)],
                    out_hbm.at[cid, pl.ds(sid * RPT, RPT)])


BPW = B // NW


@functools.partial(
    pl.kernel,
    mesh=_mesh,
    compiler_params=pltpu.CompilerParams(use_tc_tiling_on_sc=False),
    out_type=(jax.ShapeDtypeStruct((B, D), jnp.float32),
              jax.ShapeDtypeStruct((B, D), jnp.float32)),
    scratch_types=[
        pltpu.VMEM((BPW,), jnp.int32),
        pltpu.VMEM((BPW,), jnp.int32),
        pltpu.VMEM((BPW, D), jnp.float32),
        pltpu.VMEM((BPW, D), jnp.float32),
        pltpu.SemaphoreType.DMA,
        pltpu.SemaphoreType.DMA,
    ],
)
def _sc_lookup(x_hbm, r_hbm, sub_hbm, rel_hbm, sub_out, rel_out,
               sub_v, rel_v, buf_x, buf_r, sem_x, sem_r):
    wid = lax.axis_index("c") * NS + lax.axis_index("s")
    base = wid * BPW
    pltpu.sync_copy(sub_hbm.at[pl.ds(base, BPW)], sub_v)
    pltpu.sync_copy(rel_hbm.at[pl.ds(base, BPW)], rel_v)
    cp_x = pltpu.async_copy(x_hbm.at[sub_v], buf_x, sem_x)
    cp_r = pltpu.async_copy(r_hbm.at[rel_v], buf_r, sem_r)
    cp_x.wait()
    cp_r.wait()
    pltpu.sync_copy(buf_x, sub_out.at[pl.ds(base, BPW)])
    pltpu.sync_copy(buf_r, rel_out.at[pl.ds(base, BPW)])


def _tc_dense(acc_ref, embed_ref, w_ref, wl_ref, b_ref, rel_ref, wr_ref,
              x_ref, r_ref):
    pre = acc_ref[0, :N_ENT, :D] + acc_ref[1, :N_ENT, :D]
    deg = acc_ref[0, :N_ENT, D:D + 1] + acc_ref[1, :N_ENT, D:D + 1]
    norm = 1.0 / jnp.maximum(deg, 1.0)
    agg = jnp.dot(pre, w_ref[...], preferred_element_type=jnp.float32) * norm
    loop = jnp.dot(embed_ref[...], wl_ref[...], preferred_element_type=jnp.float32)
    x_ref[...] = jnp.tanh(agg + loop + b_ref[...])
    r_ref[...] = jnp.dot(rel_ref[...], wr_ref[...], preferred_element_type=jnp.float32)


_tc_dense_call = pl.pallas_call(
    _tc_dense,
    out_shape=(jax.ShapeDtypeStruct((N_ENT, D), jnp.float32),
               jax.ShapeDtypeStruct((R2, D), jnp.float32)),
)


def kernel(init_embed, init_rel, W, W_loop, W_rel, b, edge_index, edge_type,
           sub, rel):
    f32 = jnp.float32
    eidx = edge_index.astype(jnp.int32).reshape(2, NW, NJ, CHUNK)
    typ = edge_type.astype(jnp.int32).reshape(NW, NJ, CHUNK)
    idx_packed = jnp.stack([eidx[0], eidx[1], typ], axis=2)
    embed_ext = jnp.concatenate(
        [init_embed.astype(f32),
         jnp.ones((N_ENT, 1), f32),
         jnp.zeros((N_ENT, EXT - D - 1), f32)], axis=1)
    negrel_ext = jnp.concatenate(
        [-init_rel.astype(f32), jnp.zeros((R2, EXT - D), f32)], axis=1)
    zeros_blk = jnp.zeros((RPT, EXT), f32)

    acc2 = _sc_aggregate(embed_ext, negrel_ext, idx_packed, zeros_blk)
    x, r = _tc_dense_call(acc2, init_embed, W, W_loop,
                          b.reshape(1, D), init_rel, W_rel)
    sub_emb, rel_emb = _sc_lookup(x, r, sub.astype(jnp.int32),
                                  rel.astype(jnp.int32))
    return (sub_emb, rel_emb, x)


# speedup vs baseline: 7.7991x; 1.0061x over previous
import functools

import jax
import jax.numpy as jnp
from jax import lax
from jax.experimental import pallas as pl
from jax.experimental.pallas import tpu as pltpu
from jax.experimental.pallas import tpu_sc as plsc

N_ENT = 10000
D = 128
EXT = 144
E = 320000
B = 4096
R2 = 200

_info = plsc.get_sparse_core_info()
NC = _info.num_cores
NS = _info.num_subcores
NW = NC * NS
EPW = E // NW
CHUNK = 100
NJ = EPW // CHUNK
JBLK = 10
NBLK = NJ // JBLK
N_PAD = 10240
RPT = N_PAD // NS

_mesh = plsc.VectorSubcoreMesh(core_axis_name="c", subcore_axis_name="s")


@functools.partial(
    pl.kernel,
    mesh=_mesh,
    compiler_params=pltpu.CompilerParams(use_tc_tiling_on_sc=False),
    out_type=jax.ShapeDtypeStruct((NC, N_PAD, EXT), jnp.float32),
    scratch_types=[
        pltpu.VMEM((2, JBLK, 3, CHUNK), jnp.int32),
        pltpu.VMEM((CHUNK, EXT), jnp.float32),
        pltpu.VMEM((CHUNK, EXT), jnp.float32),
        pltpu.VMEM_SHARED((N_PAD, EXT), jnp.float32),
        pltpu.VMEM_SHARED((R2, EXT), jnp.float32),
        pltpu.SemaphoreType.DMA,
        pltpu.SemaphoreType.DMA,
        pltpu.SemaphoreType.DMA,
        pltpu.SemaphoreType.DMA,
        pltpu.SemaphoreType.DMA,
        pltpu.SemaphoreType.DMA,
    ],
)
def _sc_aggregate(embed_ext_hbm, negrel_ext_hbm, idx_hbm, zeros_hbm, out_hbm,
                  idx_v, buf0, buf1, acc, negrel_sp,
                  sga0, sga1, sgb0, sgb1, ssc0, ssc1):
    cid = lax.axis_index("c")
    sid = lax.axis_index("s")
    wid = cid * NS + sid
    bufs = (buf0, buf1)
    sgas = (sga0, sga1)
    sgbs = (sgb0, sgb1)
    sscs = (ssc0, ssc1)
    pltpu.sync_copy(zeros_hbm, acc.at[pl.ds(sid * RPT, RPT)])

    @pl.when(sid == 0)
    def _stage_rel():
        pltpu.sync_copy(negrel_ext_hbm, negrel_sp)

    plsc.subcore_barrier()

    def block(t, carry):
        tb = lax.rem(t, 2)
        pltpu.sync_copy(idx_hbm.at[wid, pl.ds(t * JBLK, JBLK)], idx_v.at[tb])

        def pair(p, c2):
            nonfirst = jnp.logical_or(t > 0, p > 0)
            for s in range(2):
                j = 2 * p + s

                @pl.when(nonfirst)
                def _drain(s=s, j=j):
                    pltpu.make_async_copy(
                        bufs[s], acc.at[idx_v.at[tb, j, 1]], sscs[s]).wait()

                pltpu.async_copy(embed_ext_hbm.at[idx_v.at[tb, j, 0]],
                                 bufs[s], sgas[s])
            for s in range(2):
                j = 2 * p + s
                pltpu.make_async_copy(embed_ext_hbm.at[idx_v.at[tb, j, 0]],
                                      bufs[s], sgas[s]).wait()
                pltpu.async_copy(negrel_sp.at[idx_v.at[tb, j, 2]],
                                 bufs[s], sgbs[s], add=True)
            for s in range(2):
                j = 2 * p + s
                pltpu.make_async_copy(negrel_sp.at[idx_v.at[tb, j, 2]],
                                      bufs[s], sgbs[s]).wait()
                pltpu.async_copy(bufs[s], acc.at[idx_v.at[tb, j, 1]],
                                 sscs[s], add=True)
            return c2

        lax.fori_loop(0, JBLK // 2, pair, 0)
        return carry

    lax.fori_loop(0, NBLK, block, 0)
    for s in range(2):
        pltpu.make_async_copy(bufs[s], acc.at[idx_v.at[0, 0, 1]],
                              sscs[s]).wait()
    plsc.subcore_barrier()
    pltpu.sync_copy(acc.at[pl.ds(sid * RPT, RPT)],
                    out_hbm.at[cid, pl.ds(sid * RPT, RPT)])


BPW = B // NW


@functools.partial(
    pl.kernel,
    mesh=_mesh,
    compiler_params=pltpu.CompilerParams(use_tc_tiling_on_sc=False),
    out_type=(jax.ShapeDtypeStruct((B, D), jnp.float32),
              jax.ShapeDtypeStruct((B, D), jnp.float32)),
    scratch_types=[
        pltpu.VMEM((BPW,), jnp.int32),
        pltpu.VMEM((BPW,), jnp.int32),
        pltpu.VMEM((BPW, D), jnp.float32),
        pltpu.VMEM((BPW, D), jnp.float32),
        pltpu.SemaphoreType.DMA,
        pltpu.SemaphoreType.DMA,
    ],
)
def _sc_lookup(x_hbm, r_hbm, sub_hbm, rel_hbm, sub_out, rel_out,
               sub_v, rel_v, buf_x, buf_r, sem_x, sem_r):
    wid = lax.axis_index("c") * NS + lax.axis_index("s")
    base = wid * BPW
    pltpu.sync_copy(sub_hbm.at[pl.ds(base, BPW)], sub_v)
    pltpu.sync_copy(rel_hbm.at[pl.ds(base, BPW)], rel_v)
    cp_x = pltpu.async_copy(x_hbm.at[sub_v], buf_x, sem_x)
    cp_r = pltpu.async_copy(r_hbm.at[rel_v], buf_r, sem_r)
    cp_x.wait()
    cp_r.wait()
    pltpu.sync_copy(buf_x, sub_out.at[pl.ds(base, BPW)])
    pltpu.sync_copy(buf_r, rel_out.at[pl.ds(base, BPW)])


def _tc_dense(acc_ref, embed_ref, w_ref, wl_ref, b_ref, rel_ref, wr_ref,
              x_ref, r_ref):
    pre = acc_ref[0, :N_ENT, :D] + acc_ref[1, :N_ENT, :D]
    deg = acc_ref[0, :N_ENT, D:D + 1] + acc_ref[1, :N_ENT, D:D + 1]
    norm = 1.0 / jnp.maximum(deg, 1.0)
    agg = jnp.dot(pre, w_ref[...], preferred_element_type=jnp.float32) * norm
    loop = jnp.dot(embed_ref[...], wl_ref[...], preferred_element_type=jnp.float32)
    x_ref[...] = jnp.tanh(agg + loop + b_ref[...])
    r_ref[...] = jnp.dot(rel_ref[...], wr_ref[...], preferred_element_type=jnp.float32)


_tc_dense_call = pl.pallas_call(
    _tc_dense,
    out_shape=(jax.ShapeDtypeStruct((N_ENT, D), jnp.float32),
               jax.ShapeDtypeStruct((R2, D), jnp.float32)),
)


def kernel(init_embed, init_rel, W, W_loop, W_rel, b, edge_index, edge_type,
           sub, rel):
    f32 = jnp.float32
    eidx = edge_index.astype(jnp.int32).reshape(2, NW, NJ, CHUNK)
    typ = edge_type.astype(jnp.int32).reshape(NW, NJ, CHUNK)
    idx_packed = jnp.stack([eidx[0], eidx[1], typ], axis=2)
    embed_ext = jnp.concatenate(
        [init_embed.astype(f32),
         jnp.ones((N_ENT, 1), f32),
         jnp.zeros((N_ENT, EXT - D - 1), f32)], axis=1)
    negrel_ext = jnp.concatenate(
        [-init_rel.astype(f32), jnp.zeros((R2, EXT - D), f32)], axis=1)
    zeros_blk = jnp.zeros((RPT, EXT), f32)

    acc2 = _sc_aggregate(embed_ext, negrel_ext, idx_packed, zeros_blk)
    x, r = _tc_dense_call(acc2, init_embed, W, W_loop,
                          b.reshape(1, D), init_rel, W_rel)
    sub_emb, rel_emb = _sc_lookup(x, r, sub.astype(jnp.int32),
                                  rel.astype(jnp.int32))
    return (sub_emb, rel_emb, x)
